# baseline (speedup 1.0000x reference)
import functools

import jax
import jax.numpy as jnp
from jax import lax
from jax.experimental import pallas as pl
from jax.experimental.pallas import tpu as pltpu
from jax.experimental.pallas import tpu_sc as plsc

N_AG = 10000
N_AB = 2000
D_IN = 128
D_HID = 128
D_OUT = 64
D = 128

NPG = 10240
NPB = 2048
EPG = 163840
EPB = 32768

NC = 2
NS = 16
NW = NC * NS
K = 128
BLK = 1024
OBLK = 400
CBLK = 2000


_CHUNKS_G = EPG // NW // K
_CHUNKS_B = EPB // NW // K


def _seg_pass(src_v, dst_v, tab_hbm, zero_hbm, out_hbm, rows_v, acc_sh,
              sem_g, sem_s, n_pad, chunks, c, s):
  rpt = n_pad // NS
  r0 = s * rpt
  pltpu.sync_copy(zero_hbm.at[pl.ds(r0, rpt)], acc_sh.at[pl.ds(r0, rpt)])
  plsc.subcore_barrier()

  pltpu.async_copy(tab_hbm.at[src_v.at[0]], rows_v.at[0], sem_g)

  def body(i, carry):
    b = lax.rem(i, 2)

    @pl.when(i >= 1)
    def _():
      pltpu.make_async_copy(
          rows_v.at[1 - b], acc_sh.at[dst_v.at[i - 1]], sem_s).wait()

    @pl.when(i + 1 < chunks)
    def _():
      pltpu.async_copy(tab_hbm.at[src_v.at[i + 1]], rows_v.at[1 - b], sem_g)

    pltpu.make_async_copy(tab_hbm.at[src_v.at[i]], rows_v.at[b], sem_g).wait()
    pltpu.async_copy(rows_v.at[b], acc_sh.at[dst_v.at[i]], sem_s, add=True)
    return carry

  lax.fori_loop(0, chunks, body, 0)
  pltpu.make_async_copy(
      rows_v.at[(chunks - 1) % 2],
      acc_sh.at[dst_v.at[chunks - 1]], sem_s).wait()
  plsc.subcore_barrier()
  pltpu.sync_copy(acc_sh.at[pl.ds(r0, rpt)], out_hbm.at[c, pl.ds(r0, rpt)])


@functools.lru_cache(maxsize=None)
def _make_segsum(n_pad, e_pad):
  chunks = e_pad // NW // K
  mesh = plsc.VectorSubcoreMesh(core_axis_name="c", subcore_axis_name="s")

  @functools.partial(
      pl.kernel,
      mesh=mesh,
      out_type=jax.ShapeDtypeStruct((NC, n_pad, D), jnp.float32),
      scratch_types=[
          pltpu.VMEM((chunks, K), jnp.int32),
          pltpu.VMEM((chunks, K), jnp.int32),
          pltpu.VMEM((2, K, D), jnp.float32),
          pltpu.VMEM_SHARED((n_pad, D), jnp.float32),
          pltpu.SemaphoreType.DMA,
          pltpu.SemaphoreType.DMA,
      ],
  )
  def seg(src_hbm, dst_hbm, tab_hbm, zero_hbm, out_hbm,
          src_v, dst_v, rows_v, acc_sh, sem_g, sem_s):
    c = lax.axis_index("c")
    s = lax.axis_index("s")
    wid = s * NC + c
    pltpu.sync_copy(src_hbm.at[wid], src_v)
    pltpu.sync_copy(dst_hbm.at[wid], dst_v)
    _seg_pass(src_v, dst_v, tab_hbm, zero_hbm, out_hbm, rows_v, acc_sh,
              sem_g, sem_s, n_pad, chunks, c, s)

  return seg


@functools.lru_cache(maxsize=None)
def _make_deghist(n_pad, e_pad):
  groups = e_pad // NW // 16
  mesh = plsc.VectorSubcoreMesh(core_axis_name="c", subcore_axis_name="s")

  @functools.partial(
      pl.kernel,
      mesh=mesh,
      out_type=jax.ShapeDtypeStruct((NW, n_pad), jnp.float32),
      scratch_types=[
          pltpu.VMEM((groups, 16), jnp.int32),
          pltpu.VMEM((n_pad,), jnp.float32),
      ],
      compiler_params=pltpu.CompilerParams(needs_layout_passes=False),
  )
  def deg(dst_hbm, out_hbm, dst_v, hist):
    c = lax.axis_index("c")
    s = lax.axis_index("s")
    wid = s * NC + c
    pltpu.sync_copy(dst_hbm.at[wid], dst_v)

    def zero(i, carry):
      hist[pl.ds(i * 16, 16)] = jnp.zeros((16,), jnp.float32)
      return carry

    lax.fori_loop(0, n_pad // 16, zero, 0)

    ones = jnp.ones((16,), jnp.float32)

    def body(i, carry):
      plsc.addupdate_scatter(hist, [dst_v[i]], ones)
      return carry

    lax.fori_loop(0, groups, body, 0)
    pltpu.sync_copy(hist, out_hbm.at[wid])

  return deg


def _dis_body(dp_ref, out_ref):
  deg = jnp.sum(dp_ref[...], axis=0) + 1.0
  out_ref[...] = lax.rsqrt(deg)[:, None]


def _dis(dp, n_pad):
  return pl.pallas_call(
      _dis_body,
      grid=(n_pad // BLK,),
      in_specs=[pl.BlockSpec((NW, BLK), lambda i: (0, i))],
      out_specs=pl.BlockSpec((BLK, 1), lambda i: (i, 0)),
      out_shape=jax.ShapeDtypeStruct((n_pad, 1), jnp.float32),
  )(dp)


def _mm_scale_body(x_ref, w_ref, dis_ref, out_ref):
  h = jnp.dot(x_ref[...], w_ref[...], preferred_element_type=jnp.float32)
  out_ref[...] = h * dis_ref[...]


def _mm_scale(x, w, dis, n_pad):
  return pl.pallas_call(
      _mm_scale_body,
      grid=(n_pad // BLK,),
      in_specs=[
          pl.BlockSpec((BLK, D), lambda i: (i, 0)),
          pl.BlockSpec((D, D), lambda i: (0, 0)),
          pl.BlockSpec((BLK, 1), lambda i: (i, 0)),
      ],
      out_specs=pl.BlockSpec((BLK, D), lambda i: (i, 0)),
      out_shape=jax.ShapeDtypeStruct((n_pad, D), jnp.float32),
  )(x, w, dis)


def _combine_mm_body(acc_ref, hs_ref, dis_ref, b_ref, w_ref, out_ref):
  dis = dis_ref[...]
  s = acc_ref[0] + acc_ref[1] + hs_ref[...]
  h = jnp.maximum(s * dis + b_ref[...], 0.0)
  h2 = jnp.dot(h, w_ref[...], preferred_element_type=jnp.float32)
  out_ref[...] = h2 * dis


def _combine_mm(acc, hs, dis, b, w, n_pad):
  return pl.pallas_call(
      _combine_mm_body,
      grid=(n_pad // BLK,),
      in_specs=[
          pl.BlockSpec((NC, BLK, D), lambda i: (0, i, 0)),
          pl.BlockSpec((BLK, D), lambda i: (i, 0)),
          pl.BlockSpec((BLK, 1), lambda i: (i, 0)),
          pl.BlockSpec((1, D), lambda i: (0, 0)),
          pl.BlockSpec((D, D), lambda i: (0, 0)),
      ],
      out_specs=pl.BlockSpec((BLK, D), lambda i: (i, 0)),
      out_shape=jax.ShapeDtypeStruct((n_pad, D), jnp.float32),
  )(acc, hs, dis, b, w)


def _combine_body(acc_ref, hs_ref, dis_ref, b_ref, out_ref):
  s = acc_ref[0] + acc_ref[1] + hs_ref[...]
  out_ref[...] = jnp.maximum(s * dis_ref[...] + b_ref[...], 0.0)[:, :D_OUT]


def _combine(acc, hs, dis, b, n, n_pad):
  return pl.pallas_call(
      _combine_body,
      grid=(n // CBLK,),
      in_specs=[
          pl.BlockSpec((NC, CBLK, D), lambda i: (0, i, 0)),
          pl.BlockSpec((CBLK, D), lambda i: (i, 0)),
          pl.BlockSpec((CBLK, 1), lambda i: (i, 0)),
          pl.BlockSpec((1, D), lambda i: (0, 0)),
      ],
      out_specs=pl.BlockSpec((CBLK, D_OUT), lambda i: (i, 0)),
      out_shape=jax.ShapeDtypeStruct((n, D_OUT), jnp.float32),
  )(acc, hs, dis, b)


def _logits(ag_ref, wint_ref, ab_ref):
  a2 = jnp.dot(ag_ref[...], wint_ref[...], preferred_element_type=jnp.float32)
  return lax.dot_general(
      a2, ab_ref[...],
      dimension_numbers=(((1,), (1,)), ((), ())),
      preferred_element_type=jnp.float32,
  )


def _decoder_body(ag_ref, wint_ref, ab_ref, lg_ref, epi_ref):
  logits = _logits(ag_ref, wint_ref, ab_ref)
  lg_ref[...] = logits
  epi_ref[...] = jax.nn.sigmoid(jnp.max(logits, axis=1))[:, None]


def _decoder(ag, wint, ab):
  logits, epi = pl.pallas_call(
      _decoder_body,
      grid=(N_AG // OBLK,),
      in_specs=[
          pl.BlockSpec((OBLK, D_OUT), lambda i: (i, 0)),
          pl.BlockSpec((D_OUT, D_OUT), lambda i: (0, 0)),
          pl.BlockSpec((N_AB, D_OUT), lambda i: (0, 0)),
      ],
      out_specs=[
          pl.BlockSpec((OBLK, N_AB), lambda i: (i, 0)),
          pl.BlockSpec((OBLK, 1), lambda i: (i, 0)),
      ],
      out_shape=[
          jax.ShapeDtypeStruct((N_AG, N_AB), jnp.float32),
          jax.ShapeDtypeStruct((N_AG, 1), jnp.float32),
      ],
  )(ag, wint, ab)
  return jax.nn.sigmoid(logits), epi


def _pad_edges(edge_index, n, n_pad, e_pad):
  src = edge_index[0].astype(jnp.int32)
  dst = edge_index[1].astype(jnp.int32)
  e = src.shape[0]
  e_per_w = e_pad // NW
  fill = n + (jnp.arange(e_pad - e, dtype=jnp.int32) % (n_pad - n))
  src_p = jnp.concatenate([src, fill])
  dst_p = jnp.concatenate([dst, fill])
  return (src_p.reshape(NW, e_per_w // K, K),
          dst_p.reshape(NW, e_per_w // K, K),
          dst_p.reshape(NW, e_per_w // 16, 16))


def _pad128(w):
  return jnp.zeros((D, D), jnp.float32).at[:w.shape[0], :w.shape[1]].set(w)


def _padb(b):
  return jnp.zeros((1, D), jnp.float32).at[0, :b.shape[0]].set(b)


@jax.jit
def kernel(x_g, edge_index_g, x_b, edge_index_b,
           W1g, b1g, W2g, b2g, W1b, b1b, W2b, b2b, Wint):
  src_g, dst_g, dst16_g = _pad_edges(edge_index_g, N_AG, NPG, EPG)
  src_b, dst_b, dst16_b = _pad_edges(edge_index_b, N_AB, NPB, EPB)
  xg_p = jnp.zeros((NPG, D_IN), jnp.float32).at[:N_AG].set(x_g)
  xb_p = jnp.zeros((NPB, D_IN), jnp.float32).at[:N_AB].set(x_b)
  z = jnp.zeros((NPG, D), jnp.float32)

  seg_g = _make_segsum(NPG, EPG)
  seg_b = _make_segsum(NPB, EPB)
  dpg = _make_deghist(NPG, EPG)(dst16_g)
  dpb = _make_deghist(NPB, EPB)(dst16_b)
  disg = _dis(dpg, NPG)
  disb = _dis(dpb, NPB)
  hs1g = _mm_scale(xg_p, W1g, disg, NPG)
  hs1b = _mm_scale(xb_p, W1b, disb, NPB)
  acc1g = seg_g(src_g, dst_g, hs1g, z)
  acc1b = seg_b(src_b, dst_b, hs1b, z[:NPB])
  hs2g = _combine_mm(acc1g, hs1g, disg, _padb(b1g), _pad128(W2g), NPG)
  hs2b = _combine_mm(acc1b, hs1b, disb, _padb(b1b), _pad128(W2b), NPB)
  acc2g = seg_g(src_g, dst_g, hs2g, z)
  acc2b = seg_b(src_b, dst_b, hs2b, z[:NPB])
  ag_emb = _combine(acc2g, hs2g, disg, _padb(b2g), N_AG, NPG)
  ab_emb = _combine(acc2b, hs2b, disb, _padb(b2b), N_AB, NPB)

  ip, epi_p = _decoder(ag_emb, Wint, ab_emb)

  return (ag_emb, ab_emb, ip, epi_p[:, 0])


# speedup vs baseline: 1.8254x; 1.8254x over previous
import functools

import jax
import jax.numpy as jnp
from jax import lax
from jax.experimental import pallas as pl
from jax.experimental.pallas import tpu as pltpu
from jax.experimental.pallas import tpu_sc as plsc

N_AG = 10000
N_AB = 2000
D_IN = 128
D_HID = 128
D_OUT = 64
D = 128

NPG = 10240
NPB = 2048
EPG = 163840
EPB = 32768

NC = 2
NS = 16
NW = NC * NS
K = 128
BLK = 1024
OBLK = 400
CBLK = 2000


_CHUNKS_G = EPG // NW // K
_CHUNKS_B = EPB // NW // K


def _seg_pass(src_v, dst_v, tab_hbm, zero_hbm, out_hbm, rows_v, acc_sh,
              sem_g, sem_s, n_pad, chunks, c, s):
  rpt = n_pad // NS
  r0 = s * rpt
  pltpu.sync_copy(zero_hbm.at[pl.ds(r0, rpt)], acc_sh.at[pl.ds(r0, rpt)])
  plsc.subcore_barrier()

  pltpu.async_copy(tab_hbm.at[src_v.at[0]], rows_v.at[0], sem_g)

  def body(i, carry):
    b = lax.rem(i, 2)

    @pl.when(i >= 1)
    def _():
      pltpu.make_async_copy(
          rows_v.at[1 - b], acc_sh.at[dst_v.at[i - 1]], sem_s).wait()

    @pl.when(i + 1 < chunks)
    def _():
      pltpu.async_copy(tab_hbm.at[src_v.at[i + 1]], rows_v.at[1 - b], sem_g)

    pltpu.make_async_copy(tab_hbm.at[src_v.at[i]], rows_v.at[b], sem_g).wait()
    pltpu.async_copy(rows_v.at[b], acc_sh.at[dst_v.at[i]], sem_s, add=True)
    return carry

  lax.fori_loop(0, chunks, body, 0)
  pltpu.make_async_copy(
      rows_v.at[(chunks - 1) % 2],
      acc_sh.at[dst_v.at[chunks - 1]], sem_s).wait()
  plsc.subcore_barrier()
  pltpu.sync_copy(acc_sh.at[pl.ds(r0, rpt)], out_hbm.at[c, pl.ds(r0, rpt)])


@functools.lru_cache(maxsize=None)
def _make_segsum(n_pad, e_pad):
  chunks = e_pad // NW // K
  mesh = plsc.VectorSubcoreMesh(core_axis_name="c", subcore_axis_name="s")

  @functools.partial(
      pl.kernel,
      mesh=mesh,
      out_type=jax.ShapeDtypeStruct((NC, n_pad, D), jnp.float32),
      scratch_types=[
          pltpu.VMEM((chunks, K), jnp.int32),
          pltpu.VMEM((chunks, K), jnp.int32),
          pltpu.VMEM((2, K, D), jnp.float32),
          pltpu.VMEM_SHARED((n_pad, D), jnp.float32),
          pltpu.SemaphoreType.DMA,
          pltpu.SemaphoreType.DMA,
      ],
  )
  def seg(src_hbm, dst_hbm, tab_hbm, zero_hbm, out_hbm,
          src_v, dst_v, rows_v, acc_sh, sem_g, sem_s):
    c = lax.axis_index("c")
    s = lax.axis_index("s")
    wid = s * NC + c
    pltpu.sync_copy(src_hbm.at[wid], src_v)
    pltpu.sync_copy(dst_hbm.at[wid], dst_v)
    _seg_pass(src_v, dst_v, tab_hbm, zero_hbm, out_hbm, rows_v, acc_sh,
              sem_g, sem_s, n_pad, chunks, c, s)

  return seg


@functools.lru_cache(maxsize=None)
def _make_deghist(n_pad, e_pad):
  groups = e_pad // NW // 16
  mesh = plsc.VectorSubcoreMesh(core_axis_name="c", subcore_axis_name="s")

  @functools.partial(
      pl.kernel,
      mesh=mesh,
      out_type=jax.ShapeDtypeStruct((NW, n_pad), jnp.float32),
      scratch_types=[
          pltpu.VMEM((groups, 16), jnp.int32),
          pltpu.VMEM((n_pad,), jnp.float32),
      ],
      compiler_params=pltpu.CompilerParams(needs_layout_passes=False),
  )
  def deg(dst_hbm, out_hbm, dst_v, hist):
    c = lax.axis_index("c")
    s = lax.axis_index("s")
    wid = s * NC + c
    pltpu.sync_copy(dst_hbm.at[wid], dst_v)

    def zero(i, carry):
      hist[pl.ds(i * 16, 16)] = jnp.zeros((16,), jnp.float32)
      return carry

    lax.fori_loop(0, n_pad // 16, zero, 0)

    ones = jnp.ones((16,), jnp.float32)

    def body(i, carry):
      plsc.addupdate_scatter(hist, [dst_v[i]], ones)
      return carry

    lax.fori_loop(0, groups, body, 0)
    pltpu.sync_copy(hist, out_hbm.at[wid])

  return deg


def _dis_body(dp_ref, out_ref):
  deg = jnp.sum(dp_ref[...], axis=0) + 1.0
  out_ref[...] = lax.rsqrt(deg)[:, None]


def _dis(dp, n_pad):
  return pl.pallas_call(
      _dis_body,
      grid=(n_pad // BLK,),
      in_specs=[pl.BlockSpec((NW, BLK), lambda i: (0, i))],
      out_specs=pl.BlockSpec((BLK, 1), lambda i: (i, 0)),
      out_shape=jax.ShapeDtypeStruct((n_pad, 1), jnp.float32),
  )(dp)


def _mm_scale_body(x_ref, w_ref, dis_ref, out_ref):
  h = jnp.dot(x_ref[...], w_ref[...], preferred_element_type=jnp.float32)
  out_ref[...] = h * dis_ref[...]


def _mm_scale(x, w, dis, n_pad):
  return pl.pallas_call(
      _mm_scale_body,
      grid=(n_pad // BLK,),
      in_specs=[
          pl.BlockSpec((BLK, D), lambda i: (i, 0)),
          pl.BlockSpec((D, D), lambda i: (0, 0)),
          pl.BlockSpec((BLK, 1), lambda i: (i, 0)),
      ],
      out_specs=pl.BlockSpec((BLK, D), lambda i: (i, 0)),
      out_shape=jax.ShapeDtypeStruct((n_pad, D), jnp.float32),
  )(x, w, dis)


def _combine_mm_body(acc_ref, hs_ref, dis_ref, b_ref, w_ref, out_ref):
  dis = dis_ref[...]
  s = acc_ref[0] + acc_ref[1] + hs_ref[...]
  h = jnp.maximum(s * dis + b_ref[...], 0.0)
  h2 = jnp.dot(h, w_ref[...], preferred_element_type=jnp.float32)
  out_ref[...] = h2 * dis


def _combine_mm(acc, hs, dis, b, w, n_pad):
  return pl.pallas_call(
      _combine_mm_body,
      grid=(n_pad // BLK,),
      in_specs=[
          pl.BlockSpec((NC, BLK, D), lambda i: (0, i, 0)),
          pl.BlockSpec((BLK, D), lambda i: (i, 0)),
          pl.BlockSpec((BLK, 1), lambda i: (i, 0)),
          pl.BlockSpec((1, D), lambda i: (0, 0)),
          pl.BlockSpec((D, D), lambda i: (0, 0)),
      ],
      out_specs=pl.BlockSpec((BLK, D), lambda i: (i, 0)),
      out_shape=jax.ShapeDtypeStruct((n_pad, D), jnp.float32),
  )(acc, hs, dis, b, w)


def _combine_body(acc_ref, hs_ref, dis_ref, b_ref, out_ref):
  s = acc_ref[0] + acc_ref[1] + hs_ref[...]
  out_ref[...] = jnp.maximum(s * dis_ref[...] + b_ref[...], 0.0)[:, :D_OUT]


def _combine(acc, hs, dis, b, n, n_pad):
  return pl.pallas_call(
      _combine_body,
      grid=(n // CBLK,),
      in_specs=[
          pl.BlockSpec((NC, CBLK, D), lambda i: (0, i, 0)),
          pl.BlockSpec((CBLK, D), lambda i: (i, 0)),
          pl.BlockSpec((CBLK, 1), lambda i: (i, 0)),
          pl.BlockSpec((1, D), lambda i: (0, 0)),
      ],
      out_specs=pl.BlockSpec((CBLK, D_OUT), lambda i: (i, 0)),
      out_shape=jax.ShapeDtypeStruct((n, D_OUT), jnp.float32),
  )(acc, hs, dis, b)


def _logits(ag_ref, wint_ref, ab_ref):
  a2 = jnp.dot(ag_ref[...], wint_ref[...], preferred_element_type=jnp.float32)
  return lax.dot_general(
      a2, ab_ref[...],
      dimension_numbers=(((1,), (1,)), ((), ())),
      preferred_element_type=jnp.float32,
  )


def _decoder_body(ag_ref, wint_ref, ab_ref, ip_ref, epi_ref):
  logits = _logits(ag_ref, wint_ref, ab_ref)
  ip_ref[...] = jax.nn.sigmoid(logits)
  epi_ref[...] = jax.nn.sigmoid(jnp.max(logits, axis=1))[:, None]


def _decoder(ag, wint, ab):
  return pl.pallas_call(
      _decoder_body,
      grid=(N_AG // OBLK,),
      in_specs=[
          pl.BlockSpec((OBLK, D_OUT), lambda i: (i, 0)),
          pl.BlockSpec((D_OUT, D_OUT), lambda i: (0, 0)),
          pl.BlockSpec((N_AB, D_OUT), lambda i: (0, 0)),
      ],
      out_specs=[
          pl.BlockSpec((OBLK, N_AB), lambda i: (i, 0)),
          pl.BlockSpec((OBLK, 1), lambda i: (i, 0)),
      ],
      out_shape=[
          jax.ShapeDtypeStruct((N_AG, N_AB), jnp.float32),
          jax.ShapeDtypeStruct((N_AG, 1), jnp.float32),
      ],
  )(ag, wint, ab)


def _pad_edges(edge_index, n, n_pad, e_pad):
  src = edge_index[0].astype(jnp.int32)
  dst = edge_index[1].astype(jnp.int32)
  e = src.shape[0]
  e_per_w = e_pad // NW
  fill = n + (jnp.arange(e_pad - e, dtype=jnp.int32) % (n_pad - n))
  src_p = jnp.concatenate([src, fill])
  dst_p = jnp.concatenate([dst, fill])
  return (src_p.reshape(NW, e_per_w // K, K),
          dst_p.reshape(NW, e_per_w // K, K),
          dst_p.reshape(NW, e_per_w // 16, 16))


def _pad128(w):
  return jnp.zeros((D, D), jnp.float32).at[:w.shape[0], :w.shape[1]].set(w)


def _padb(b):
  return jnp.zeros((1, D), jnp.float32).at[0, :b.shape[0]].set(b)


@jax.jit
def kernel(x_g, edge_index_g, x_b, edge_index_b,
           W1g, b1g, W2g, b2g, W1b, b1b, W2b, b2b, Wint):
  src_g, dst_g, dst16_g = _pad_edges(edge_index_g, N_AG, NPG, EPG)
  src_b, dst_b, dst16_b = _pad_edges(edge_index_b, N_AB, NPB, EPB)
  xg_p = jnp.zeros((NPG, D_IN), jnp.float32).at[:N_AG].set(x_g)
  xb_p = jnp.zeros((NPB, D_IN), jnp.float32).at[:N_AB].set(x_b)
  z = jnp.zeros((NPG, D), jnp.float32)

  seg_g = _make_segsum(NPG, EPG)
  seg_b = _make_segsum(NPB, EPB)
  dpg = _make_deghist(NPG, EPG)(dst16_g)
  dpb = _make_deghist(NPB, EPB)(dst16_b)
  disg = _dis(dpg, NPG)
  disb = _dis(dpb, NPB)
  hs1g = _mm_scale(xg_p, W1g, disg, NPG)
  hs1b = _mm_scale(xb_p, W1b, disb, NPB)
  acc1g = seg_g(src_g, dst_g, hs1g, z)
  acc1b = seg_b(src_b, dst_b, hs1b, z[:NPB])
  hs2g = _combine_mm(acc1g, hs1g, disg, _padb(b1g), _pad128(W2g), NPG)
  hs2b = _combine_mm(acc1b, hs1b, disb, _padb(b1b), _pad128(W2b), NPB)
  acc2g = seg_g(src_g, dst_g, hs2g, z)
  acc2b = seg_b(src_b, dst_b, hs2b, z[:NPB])
  ag_emb = _combine(acc2g, hs2g, disg, _padb(b2g), N_AG, NPG)
  ab_emb = _combine(acc2b, hs2b, disb, _padb(b2b), N_AB, NPB)

  ip, epi_p = _decoder(ag_emb, Wint, ab_emb)

  return (ag_emb, ab_emb, ip, epi_p[:, 0])
